# Initial kernel scaffold; baseline (speedup 1.0000x reference)
import jax
import jax.numpy as jnp
from jax.experimental import pallas as pl


def kernel(nfeat, edge_index, efeat, W, b):
    raise NotImplementedError("write your pallas kernel here")


# speedup vs baseline: 45.1362x; 45.1362x over previous
import functools

import jax
import jax.numpy as jnp
from jax import lax
from jax.experimental import pallas as pl
from jax.experimental.pallas import tpu as pltpu
from jax.experimental.pallas import tpu_sc as plsc

NC = 2
NS = 16
NW = NC * NS

C = 80
FA = 80


def _sc_gather_sum(nfeat2d, dst2d, src2d, zA, NP):
    N, F = nfeat2d.shape
    CPW = dst2d.shape[0] // NW
    RPT = NP // NS
    RCH = RPT // C

    mesh = plsc.VectorSubcoreMesh(core_axis_name="c", subcore_axis_name="s")

    @functools.partial(
        pl.kernel,
        out_type=jax.ShapeDtypeStruct((NC, NP, F), jnp.float32),
        mesh=mesh,
        scratch_types=[
            pltpu.VMEM((C,), jnp.int32),
            pltpu.VMEM((C,), jnp.int32),
            pltpu.VMEM((C, F), jnp.float32),
            pltpu.VMEM_SHARED((NP, F), jnp.float32),
        ],
    )
    def k1(nfeat_hbm, dst_hbm, src_hbm, zA_hbm, a_out, dstbuf, srcbuf, gbuf, accA):
        cid = lax.axis_index("c")
        sid = lax.axis_index("s")
        wid = sid * NC + cid
        r0 = sid * RPT

        pltpu.sync_copy(zA_hbm, gbuf)
        for k in range(RCH):
            pltpu.sync_copy(gbuf, accA.at[pl.ds(r0 + k * C, C)])
        plsc.subcore_barrier()

        def chunk_body(ch, carry):
            row = wid * CPW + ch
            pltpu.sync_copy(dst_hbm.at[row], dstbuf)
            pltpu.sync_copy(src_hbm.at[row], srcbuf)
            pltpu.sync_copy(nfeat_hbm.at[srcbuf], gbuf)
            pltpu.sync_copy(gbuf, accA.at[dstbuf], add=True)
            return carry

        lax.fori_loop(0, CPW, chunk_body, 0)

        plsc.subcore_barrier()
        for k in range(RCH):
            rs = r0 + k * C
            pltpu.sync_copy(accA.at[pl.ds(rs, C)], gbuf)
            pltpu.sync_copy(gbuf, a_out.at[cid, pl.ds(rs, C)])

    return k1(nfeat2d, dst2d, src2d, zA)


def _sc_efeat_deg_sum(efeat_aug, dst2d, zE, NP):
    E = efeat_aug.shape[0]
    EPW = E // NW
    CPW = EPW // C
    RPT = NP // NS
    RCH = RPT // C

    mesh = plsc.VectorSubcoreMesh(core_axis_name="c", subcore_axis_name="s")

    @functools.partial(
        pl.kernel,
        out_type=jax.ShapeDtypeStruct((NC, NP, FA), jnp.float32),
        mesh=mesh,
        scratch_types=[
            pltpu.VMEM((C,), jnp.int32),
            pltpu.VMEM((C, FA), jnp.float32),
            pltpu.VMEM_SHARED((NP, FA), jnp.float32),
        ],
    )
    def k2(efeat_hbm, dst_hbm, zE_hbm, e_out, idxbuf, ebuf, accE):
        cid = lax.axis_index("c")
        sid = lax.axis_index("s")
        wid = sid * NC + cid
        r0 = sid * RPT

        pltpu.sync_copy(zE_hbm, ebuf)
        for k in range(RCH):
            pltpu.sync_copy(ebuf, accE.at[pl.ds(r0 + k * C, C)])
        plsc.subcore_barrier()

        def chunk_body(ch, carry):
            ebase = wid * EPW + ch * C
            pltpu.sync_copy(dst_hbm.at[wid * CPW + ch], idxbuf)
            pltpu.sync_copy(efeat_hbm.at[pl.ds(ebase, C)], ebuf)
            pltpu.sync_copy(ebuf, accE.at[idxbuf], add=True)
            return carry

        lax.fori_loop(0, CPW, chunk_body, 0)

        plsc.subcore_barrier()
        for k in range(RCH):
            rs = r0 + k * C
            pltpu.sync_copy(accE.at[pl.ds(rs, C)], ebuf)
            pltpu.sync_copy(ebuf, e_out.at[cid, pl.ds(rs, C)])

    return k2(efeat_aug, dst2d, zE)


def _tc_combine(a_part, aug_part, nfeat2d, wq_bd, wk_bd, w_tile, b_tile):
    N, F = nfeat2d.shape
    F2 = w_tile.shape[0]
    BLK = 1000
    grid = (N // BLK,)

    def tc_body(aref, eref, xref, wqref, wkref, wtref, btref, oref):
        a = aref[0] + aref[1]
        aug = eref[0] + eref[1]
        es = aug[:, :F2]
        deg = aug[:, F2:F2 + 1]
        y = jnp.dot(a, wkref[...], preferred_element_type=jnp.float32)
        y = y + jnp.dot(es, wtref[...], preferred_element_type=jnp.float32)
        y = y + deg * (jnp.dot(xref[...], wqref[...],
                               preferred_element_type=jnp.float32) + btref[...])
        oref[...] = y

    return pl.pallas_call(
        tc_body,
        grid=grid,
        in_specs=[
            pl.BlockSpec((NC, BLK, F), lambda i: (0, i, 0)),
            pl.BlockSpec((NC, BLK, FA), lambda i: (0, i, 0)),
            pl.BlockSpec((BLK, F), lambda i: (i, 0)),
            pl.BlockSpec((F, F), lambda i: (0, 0)),
            pl.BlockSpec((F, F), lambda i: (0, 0)),
            pl.BlockSpec((F2, F), lambda i: (0, 0)),
            pl.BlockSpec((1, F), lambda i: (0, 0)),
        ],
        out_specs=pl.BlockSpec((BLK, F), lambda i: (i, 0)),
        out_shape=jax.ShapeDtypeStruct((N, F), jnp.float32),
    )(a_part, aug_part, nfeat2d, wq_bd, wk_bd, w_tile, b_tile)


def kernel(nfeat, edge_index, efeat, W, b):
    N, H, D = nfeat.shape
    E = edge_index.shape[1]
    F = H * D
    F2 = 2 * D
    NP = -(-N // (NS * C)) * (NS * C)

    nfeat2d = nfeat.reshape(N, F)
    src = edge_index[0]
    dst = edge_index[1]
    EPW = E // NW
    dst2d = dst.reshape(NW * (EPW // C), C).astype(jnp.int32)
    src2d = src.reshape(NW * (EPW // C), C).astype(jnp.int32)
    efeat_aug = jnp.concatenate(
        [efeat, jnp.ones((E, 1), jnp.float32),
         jnp.zeros((E, FA - F2 - 1), jnp.float32)], axis=1)
    zA = jnp.zeros((C, F), jnp.float32)
    zE = jnp.zeros((C, FA), jnp.float32)

    a_part = _sc_gather_sum(nfeat2d, dst2d, src2d, zA, NP)
    aug_part = _sc_efeat_deg_sum(efeat_aug, dst2d, zE, NP)

    Wq = W[:D]
    Wk = W[D:]
    wq_bd = jax.scipy.linalg.block_diag(*([Wq] * H))
    wk_bd = jax.scipy.linalg.block_diag(*([Wk] * H))
    w_tile = jnp.tile(W, (1, H))
    b_tile = jnp.tile(b, H)[None, :]

    out2d = _tc_combine(a_part, aug_part, nfeat2d, wq_bd, wk_bd, w_tile, b_tile)
    return out2d.reshape(N, H, D)
